# baseline (speedup 1.0000x reference)
import jax
import jax.numpy as jnp
from jax import lax
from jax.experimental import pallas as pl
from jax.experimental.pallas import tpu as pltpu
from jax.experimental.pallas import tpu_sc as plsc

_VOCAB = 1000000
_EMBED = 64
_B, _L = 1024, 200
_NW = 32
_ROWS = _B * _L
_RPW = _ROWS // _NW
_BPW = _B // _NW

_C = 16384
_LOGC = _C.bit_length() - 1
_HALF = _C // 2
_NBLK = (_VOCAB + _C - 1) // _C
_TROWS = _NBLK * _C


def _repack_kernel(src_ref, out_ref):
    t = jnp.transpose(src_ref[...])
    out_ref[:, 0:_EMBED] = t[0:_HALF, :]
    out_ref[:, _EMBED:2 * _EMBED] = t[_HALF:, :]


def _repack(tok_t):
    return pl.pallas_call(
        _repack_kernel,
        grid=(_NBLK,),
        in_specs=[pl.BlockSpec((_EMBED, _C), lambda g: (0, g))],
        out_specs=pl.BlockSpec((_HALF, 2 * _EMBED), lambda g: (g, 0)),
        out_shape=jax.ShapeDtypeStruct((_TROWS // 2, 2 * _EMBED), jnp.float32),
    )(tok_t)


def _embed_kernel(x_hbm, tok_hbm, pos_hbm, out_hbm,
                  idx_all, buf0, buf1, stage0, stage1, pos_v,
                  gsem0, gsem1, wsem0, wsem1):
    c = lax.axis_index("c")
    s = lax.axis_index("s")
    wid = s * 2 + c
    bufs = (buf0, buf1)
    stages = (stage0, stage1)
    gsems = (gsem0, gsem1)
    wsems = (wsem0, wsem1)

    pltpu.sync_copy(pos_hbm.at[pl.ds(0, _L)], pos_v)
    pltpu.sync_copy(x_hbm.at[pl.ds(wid * _RPW, _RPW)], idx_all)

    def xform(k, carry):
        sl = pl.ds(16 * k, 16)
        iv = idx_all[sl]
        rem = iv & (_C - 1)
        idx_all[sl] = ((iv >> _LOGC) << _LOGC) + ((rem & (_HALF - 1)) << 1) + (rem >> (_LOGC - 1))
        return carry

    lax.fori_loop(0, _RPW // 16, xform, 0)

    descs = {}

    def start(sc):
        b = sc & 1
        if sc >= 2:
            pltpu.make_async_copy(out_hbm.at[pl.ds(0, _L // 2)],
                                  stages[b], wsems[b]).wait()
        off = sc * _L
        descs[sc] = [
            pltpu.async_copy(
                tok_hbm.at[idx_all.at[pl.ds(off, 128)]],
                bufs[b].at[pl.ds(0, 128)], gsems[b]),
            pltpu.async_copy(
                tok_hbm.at[idx_all.at[pl.ds(off + 128, 72)]],
                bufs[b].at[pl.ds(128, 72)], gsems[b]),
        ]

    def process(sc):
        b = sc & 1
        for d in descs[sc]:
            d.wait()
        buf = bufs[b]
        stage = stages[b]

        def row_body(p, carry):
            for half in range(2):
                for jj in range(_EMBED // 16):
                    dsl = pl.ds(half * _EMBED + 16 * jj, 16)
                    stage[p, dsl] = (buf[2 * p + half, pl.ds(16 * jj, 16)]
                                     + pos_v[2 * p + half, pl.ds(16 * jj, 16)])
            return carry

        lax.fori_loop(0, _L // 2, row_body, 0)
        pltpu.async_copy(stage,
                         out_hbm.at[pl.ds((wid * _BPW + sc) * (_L // 2), _L // 2)],
                         wsems[b])

    start(0)
    for sc in range(_BPW):
        if sc + 1 < _BPW:
            start(sc + 1)
        process(sc)
    pltpu.make_async_copy(out_hbm.at[pl.ds(0, _L // 2)], stage0, wsem0).wait()
    pltpu.make_async_copy(out_hbm.at[pl.ds(0, _L // 2)], stage1, wsem1).wait()


def _embed(xf, tokc, pos_table):
    mesh = plsc.VectorSubcoreMesh(core_axis_name="c", subcore_axis_name="s")
    return pl.kernel(
        _embed_kernel,
        out_type=jax.ShapeDtypeStruct((_ROWS // 2, 2 * _EMBED), jnp.float32),
        mesh=mesh,
        scratch_types=[
            pltpu.VMEM((_RPW,), jnp.int32),
            pltpu.VMEM((_L, _EMBED), jnp.float32),
            pltpu.VMEM((_L, _EMBED), jnp.float32),
            pltpu.VMEM((_L // 2, 2 * _EMBED), jnp.float32),
            pltpu.VMEM((_L // 2, 2 * _EMBED), jnp.float32),
            pltpu.VMEM((_L, _EMBED), jnp.float32),
            pltpu.SemaphoreType.DMA,
            pltpu.SemaphoreType.DMA,
            pltpu.SemaphoreType.DMA,
            pltpu.SemaphoreType.DMA,
        ],
        compiler_params=pltpu.CompilerParams(use_tc_tiling_on_sc=False),
    )(xf, tokc, pos_table)


@jax.jit
def _run(x, tok_table, pos_table):
    xf = jnp.reshape(x, (_ROWS,)).astype(jnp.int32)
    tokw = _repack(jnp.transpose(tok_table))
    tokc = jnp.reshape(tokw, (_TROWS, _EMBED))
    out = _embed(xf, tokc, pos_table)
    return jnp.reshape(out, (_B, _L, _EMBED))


def kernel(x, tok_table, pos_table):
    return _run(x, tok_table, pos_table)
